# Initial kernel scaffold; baseline (speedup 1.0000x reference)
import jax
import jax.numpy as jnp
from jax.experimental import pallas as pl


def kernel(x, table):
    raise NotImplementedError("write your pallas kernel here")


# speedup vs baseline: 1.4390x; 1.4390x over previous
import functools

import jax
import jax.numpy as jnp
from jax import lax
from jax.experimental import pallas as pl
from jax.experimental.pallas import tpu as pltpu
from jax.experimental.pallas import tpu_sc as plsc

_NW = 32
_CH = 32


@functools.lru_cache(maxsize=None)
def _make_emb(n_total: int, d_model: int):
    per_w = n_total // _NW
    nch = per_w // _CH
    mesh = plsc.VectorSubcoreMesh(core_axis_name="c", subcore_axis_name="s")

    @functools.partial(
        pl.kernel,
        out_type=jax.ShapeDtypeStruct((n_total, d_model), jnp.float32),
        mesh=mesh,
        scratch_types=[
            pltpu.VMEM((nch, _CH), jnp.int32),
            pltpu.VMEM((_CH, d_model), jnp.float32),
            pltpu.SemaphoreType.DMA,
        ],
    )
    def emb(idx_hbm, table_hbm, out_hbm, idx_v, buf, sem):
        wid = lax.axis_index("s") * 2 + lax.axis_index("c")
        base = wid * per_w
        pltpu.sync_copy(idx_hbm.at[wid], idx_v)

        def body(j, carry):
            pltpu.async_copy(table_hbm.at[idx_v.at[j]], buf, sem).wait()
            pltpu.sync_copy(buf, out_hbm.at[pl.ds(base + j * _CH, _CH)])
            return carry

        lax.fori_loop(0, nch, body, 0)

    return emb


def kernel(x, table):
    n = x.size
    d = table.shape[1]
    idx = x.reshape(_NW, n // _NW // _CH, _CH).astype(jnp.int32)
    out = _make_emb(n, d)(idx, table)
    return out.reshape(x.shape + (d,))


# speedup vs baseline: 1.5969x; 1.1097x over previous
import functools

import jax
import jax.numpy as jnp
from jax import lax
from jax.experimental import pallas as pl
from jax.experimental.pallas import tpu as pltpu
from jax.experimental.pallas import tpu_sc as plsc

_NW = 32
_CH = 32


@functools.lru_cache(maxsize=None)
def _make_emb(n_total: int, d_model: int):
    per_w = n_total // _NW
    nch = per_w // _CH
    assert nch >= 4 and nch % 2 == 0
    mesh = plsc.VectorSubcoreMesh(core_axis_name="c", subcore_axis_name="s")

    @functools.partial(
        pl.kernel,
        out_type=jax.ShapeDtypeStruct((n_total, d_model), jnp.float32),
        mesh=mesh,
        scratch_types=[
            pltpu.VMEM((nch, _CH), jnp.int32),
            pltpu.VMEM((2, _CH, d_model), jnp.float32),
            pltpu.SemaphoreType.DMA,
            pltpu.SemaphoreType.DMA,
            pltpu.SemaphoreType.DMA,
        ],
    )
    def emb(idx_hbm, table_hbm, out_hbm, idx_v, buf, gsem, ssem0, ssem1):
        wid = lax.axis_index("s") * 2 + lax.axis_index("c")
        base = wid * per_w
        pltpu.sync_copy(idx_hbm.at[wid], idx_v)

        def gather(j, b):
            pltpu.async_copy(table_hbm.at[idx_v.at[j]], buf.at[b], gsem)

        def wait_gather():
            pltpu.make_async_copy(
                table_hbm.at[idx_v.at[0]], buf.at[0], gsem).wait()

        def scatter(j, b, sem):
            pltpu.async_copy(
                buf.at[b], out_hbm.at[pl.ds(base + j * _CH, _CH)], sem)

        def wait_scatter(sem):
            pltpu.make_async_copy(
                buf.at[0], out_hbm.at[pl.ds(base, _CH)], sem).wait()

        gather(0, 0)
        wait_gather()
        scatter(0, 0, ssem0)
        gather(1, 1)

        def body(i, carry):
            j = 1 + 2 * i
            wait_gather()
            scatter(j, 1, ssem1)
            wait_scatter(ssem0)
            gather(j + 1, 0)
            wait_gather()
            scatter(j + 1, 0, ssem0)
            wait_scatter(ssem1)
            gather(j + 2, 1)
            return carry

        lax.fori_loop(0, (nch - 2) // 2, body, 0)

        wait_gather()
        scatter(nch - 1, 1, ssem1)
        wait_scatter(ssem0)
        wait_scatter(ssem1)

    return emb


def kernel(x, table):
    n = x.size
    d = table.shape[1]
    idx = x.reshape(_NW, n // _NW // _CH, _CH).astype(jnp.int32)
    out = _make_emb(n, d)(idx, table)
    return out.reshape(x.shape + (d,))


# speedup vs baseline: 1.6588x; 1.0388x over previous
import functools

import jax
import jax.numpy as jnp
from jax import lax
from jax.experimental import pallas as pl
from jax.experimental.pallas import tpu as pltpu
from jax.experimental.pallas import tpu_sc as plsc

_NW = 32
_CH = 16
_NBUF = 4


@functools.lru_cache(maxsize=None)
def _make_emb(n_total: int, d_model: int):
    per_w = n_total // _NW
    nch = per_w // _CH
    assert nch >= 2 * _NBUF and nch % _NBUF == 0
    mesh = plsc.VectorSubcoreMesh(core_axis_name="c", subcore_axis_name="s")

    @functools.partial(
        pl.kernel,
        out_type=jax.ShapeDtypeStruct((n_total, d_model), jnp.float32),
        mesh=mesh,
        scratch_types=[
            pltpu.VMEM((nch, _CH), jnp.int32),
            pltpu.VMEM((_NBUF, _CH, d_model), jnp.float32),
        ]
        + [pltpu.SemaphoreType.DMA] * (2 * _NBUF),
    )
    def emb(idx_hbm, table_hbm, out_hbm, idx_v, buf, *sems):
        gs, ss = sems[:_NBUF], sems[_NBUF:]
        wid = lax.axis_index("s") * 2 + lax.axis_index("c")
        base = wid * per_w
        pltpu.sync_copy(idx_hbm.at[wid], idx_v)

        def gather(j, b):
            pltpu.async_copy(table_hbm.at[idx_v.at[j]], buf.at[b], gs[b])

        def wait_gather(b):
            pltpu.make_async_copy(
                table_hbm.at[idx_v.at[0]], buf.at[b], gs[b]).wait()

        def scatter(j, b):
            pltpu.async_copy(
                buf.at[b], out_hbm.at[pl.ds(base + j * _CH, _CH)], ss[b])

        def wait_scatter(b):
            pltpu.make_async_copy(
                buf.at[b], out_hbm.at[pl.ds(base, _CH)], ss[b]).wait()

        gather(0, 0)
        gather(1, 1)
        wait_gather(0)
        scatter(0, 0)
        gather(2, 2)
        wait_gather(1)
        scatter(1, 1)
        gather(3, 3)

        def body(i, carry):
            j0 = 2 + _NBUF * i
            for u in range(_NBUF):
                b = (2 + u) % _NBUF
                wait_gather(b)
                scatter(j0 + u, b)
                wait_scatter((b + 2) % _NBUF)
                gather(j0 + u + 2, (b + 2) % _NBUF)
            return carry

        lax.fori_loop(0, (nch - 4) // _NBUF, body, 0)

        wait_gather(2)
        scatter(nch - 2, 2)
        wait_gather(3)
        scatter(nch - 1, 3)
        for b in range(_NBUF):
            wait_scatter(b)

    return emb


def kernel(x, table):
    n = x.size
    d = table.shape[1]
    idx = x.reshape(_NW, n // _NW // _CH, _CH).astype(jnp.int32)
    out = _make_emb(n, d)(idx, table)
    return out.reshape(x.shape + (d,))


# speedup vs baseline: 1.6694x; 1.0064x over previous
import functools

import jax
import jax.numpy as jnp
from jax import lax
from jax.experimental import pallas as pl
from jax.experimental.pallas import tpu as pltpu
from jax.experimental.pallas import tpu_sc as plsc

_NW = 32
_CH = 16
_NBUF = 6
_K = 3


@functools.lru_cache(maxsize=None)
def _make_emb(n_total: int, d_model: int):
    per_w = n_total // _NW
    nch = per_w // _CH
    assert nch >= 2 * _NBUF
    mesh = plsc.VectorSubcoreMesh(core_axis_name="c", subcore_axis_name="s")

    @functools.partial(
        pl.kernel,
        out_type=jax.ShapeDtypeStruct((n_total, d_model), jnp.float32),
        mesh=mesh,
        scratch_types=[
            pltpu.VMEM((nch, _CH), jnp.int32),
            pltpu.VMEM((_NBUF, _CH, d_model), jnp.float32),
        ]
        + [pltpu.SemaphoreType.DMA] * (2 * _NBUF),
    )
    def emb(idx_hbm, table_hbm, out_hbm, idx_v, buf, *sems):
        gs, ss = sems[:_NBUF], sems[_NBUF:]
        wid = lax.axis_index("s") * 2 + lax.axis_index("c")
        base = wid * per_w
        pltpu.sync_copy(idx_hbm.at[wid], idx_v)

        def gather(j, b):
            pltpu.async_copy(table_hbm.at[idx_v.at[j]], buf.at[b], gs[b])

        def wait_gather(b):
            pltpu.make_async_copy(
                table_hbm.at[idx_v.at[0]], buf.at[b], gs[b]).wait()

        def scatter(j, b):
            pltpu.async_copy(
                buf.at[b], out_hbm.at[pl.ds(base + j * _CH, _CH)], ss[b])

        def wait_scatter(b):
            pltpu.make_async_copy(
                buf.at[b], out_hbm.at[pl.ds(base, _CH)], ss[b]).wait()

        def step(j, b, fresh):
            wait_gather(b)
            scatter(j, b)
            bg = (b + _K) % _NBUF
            if not fresh:
                wait_scatter(bg)
            gather(j + _K, bg)

        for j in range(_K):
            gather(j, j)
        for j in range(_NBUF - _K):
            step(j, j, fresh=True)

        steady = nch - _NBUF
        main = (steady // _NBUF) * _NBUF

        def body(i, carry):
            j0 = (_NBUF - _K) + _NBUF * i
            for u in range(_NBUF):
                step(j0 + u, (_NBUF - _K + u) % _NBUF, fresh=False)
            return carry

        lax.fori_loop(0, main // _NBUF, body, 0)

        for r in range(steady - main):
            j = (_NBUF - _K) + main + r
            step(j, j % _NBUF, fresh=False)
        for j in range(nch - _K, nch):
            wait_gather(j % _NBUF)
            scatter(j, j % _NBUF)
        for b in range(_NBUF):
            wait_scatter(b)

    return emb


def kernel(x, table):
    n = x.size
    d = table.shape[1]
    idx = x.reshape(_NW, n // _NW // _CH, _CH).astype(jnp.int32)
    out = _make_emb(n, d)(idx, table)
    return out.reshape(x.shape + (d,))


# speedup vs baseline: 1.6803x; 1.0065x over previous
import functools

import jax
import jax.numpy as jnp
from jax import lax
from jax.experimental import pallas as pl
from jax.experimental.pallas import tpu as pltpu
from jax.experimental.pallas import tpu_sc as plsc

_NW = 32
_CH = 16
_NBUF = 6
_K = 3


@functools.lru_cache(maxsize=None)
def _make_emb(n_total: int, d_model: int):
    per_w = n_total // _NW
    nch = per_w // _CH
    assert nch >= 2 * _NBUF
    mesh = plsc.VectorSubcoreMesh(core_axis_name="c", subcore_axis_name="s")

    @functools.partial(
        pl.kernel,
        out_type=jax.ShapeDtypeStruct((n_total, d_model), jnp.float32),
        mesh=mesh,
        scratch_types=[
            pltpu.VMEM((nch, _CH), jnp.int32),
            pltpu.VMEM((_NBUF, _CH, d_model), jnp.float32),
        ]
        + [pltpu.SemaphoreType.DMA] * (2 * _NBUF),
    )
    def emb(idx_hbm, table_hbm, out_hbm, idx_v, buf, *sems):
        gs, ss = sems[:_NBUF], sems[_NBUF:]
        wid = lax.axis_index("s") * 2 + lax.axis_index("c")
        base = wid * per_w
        pltpu.sync_copy(idx_hbm.at[wid], idx_v)

        def gather(j, b):
            pltpu.async_copy(table_hbm.at[idx_v.at[j]], buf.at[b], gs[b])

        def wait_gather(b):
            pltpu.make_async_copy(
                table_hbm.at[idx_v.at[0]], buf.at[b], gs[b]).wait()

        def scatter(j, b):
            pltpu.async_copy(
                buf.at[b], out_hbm.at[pl.ds(base + j * _CH, _CH)], ss[b])

        def wait_scatter(b):
            pltpu.make_async_copy(
                buf.at[b], out_hbm.at[pl.ds(base, _CH)], ss[b]).wait()

        def step(j, b, fresh):
            bg = (b + _K) % _NBUF
            if not fresh:
                wait_scatter(bg)
            gather(j + _K, bg)
            wait_gather(b)
            scatter(j, b)

        for j in range(_K):
            gather(j, j)
        for j in range(_NBUF - _K):
            step(j, j, fresh=True)

        steady = nch - _NBUF
        main = (steady // _NBUF) * _NBUF

        def body(i, carry):
            j0 = (_NBUF - _K) + _NBUF * i
            for u in range(_NBUF):
                step(j0 + u, (_NBUF - _K + u) % _NBUF, fresh=False)
            return carry

        lax.fori_loop(0, main // _NBUF, body, 0)

        for r in range(steady - main):
            j = (_NBUF - _K) + main + r
            step(j, j % _NBUF, fresh=False)
        for j in range(nch - _K, nch):
            wait_gather(j % _NBUF)
            scatter(j, j % _NBUF)
        for b in range(_NBUF):
            wait_scatter(b)

    return emb


def kernel(x, table):
    n = x.size
    d = table.shape[1]
    idx = x.reshape(_NW, n // _NW // _CH, _CH).astype(jnp.int32)
    out = _make_emb(n, d)(idx, table)
    return out.reshape(x.shape + (d,))
